# Initial kernel scaffold; baseline (speedup 1.0000x reference)
import jax
import jax.numpy as jnp
from jax.experimental import pallas as pl


def kernel(words, special_words, word_table, special_table):
    raise NotImplementedError("write your pallas kernel here")


# speedup vs baseline: 1.0087x; 1.0087x over previous
import jax
import jax.numpy as jnp
from jax import lax
from jax.experimental import pallas as pl
from jax.experimental.pallas import tpu as pltpu
from jax.experimental.pallas import tpu_sc as plsc

DIM = 300
DIM_PAD = 304
N_SPECIAL = 6
CHUNK = 128
N_WORKERS = 32


def _fasttext_body(words_hbm, sw_hbm, wt_hbm, sp_hbm, out_hbm,
                   idxw_v, idxs_v, rows_v, spbuf_v, sem):
    wid = lax.axis_index("s") * 2 + lax.axis_index("c")
    per_worker = words_hbm.shape[0] // N_WORKERS
    n_chunks = per_worker // CHUNK

    pltpu.sync_copy(sp_hbm, spbuf_v)

    @pl.loop(0, n_chunks)
    def chunk_body(c):
        base = wid * per_worker + c * CHUNK
        pltpu.sync_copy(words_hbm.at[pl.ds(base, CHUNK)], idxw_v)
        pltpu.sync_copy(sw_hbm.at[pl.ds(base, CHUNK)], idxs_v)
        pltpu.async_copy(wt_hbm.at[idxw_v], rows_v, sem).wait()

        @pl.loop(0, CHUNK // 16)
        def tok_group_body(tg):
            swv = idxs_v[pl.ds(tg * 16, 16)]
            swv = jnp.where(swv == 0, N_SPECIAL, swv)
            for k in range(16):
                sw = swv[k]
                t = tg * 16 + k

                @pl.when(sw != N_SPECIAL)
                def _():
                    for j in range(DIM_PAD // 16):
                        sp16 = spbuf_v[sw, pl.ds(j * 16, 16)]
                        plsc.addupdate(rows_v.at[t, pl.ds(j * 16, 16)], sp16)

        pltpu.sync_copy(rows_v, out_hbm.at[pl.ds(base, CHUNK)])


def kernel(words, special_words, word_table, special_table):
    B, S = words.shape
    words_f = words.reshape(-1).astype(jnp.int32)
    sw_f = special_words.reshape(-1).astype(jnp.int32)
    wt_pad = jnp.pad(word_table, ((0, 0), (0, DIM_PAD - DIM)))
    sp_pad = jnp.pad(special_table, ((0, 0), (0, DIM_PAD - DIM)))
    run = pl.kernel(
        _fasttext_body,
        out_type=jax.ShapeDtypeStruct((B * S, DIM_PAD), jnp.float32),
        mesh=plsc.VectorSubcoreMesh(core_axis_name="c", subcore_axis_name="s"),
        scratch_types=[
            pltpu.VMEM((CHUNK,), jnp.int32),
            pltpu.VMEM((CHUNK,), jnp.int32),
            pltpu.VMEM((CHUNK, DIM_PAD), jnp.float32),
            pltpu.VMEM((N_SPECIAL + 1, DIM_PAD), jnp.float32),
            pltpu.SemaphoreType.DMA,
        ],
        compiler_params=pltpu.CompilerParams(use_tc_tiling_on_sc=False),
    )
    out = run(words_f, sw_f, wt_pad, sp_pad)
    return out[:, :DIM].reshape(B, S, DIM)


# speedup vs baseline: 1.9330x; 1.9163x over previous
import jax
import jax.numpy as jnp
from jax import lax
from jax.experimental import pallas as pl
from jax.experimental.pallas import tpu as pltpu
from jax.experimental.pallas import tpu_sc as plsc

DIM = 300
DIM_PAD = 384
N_SPECIAL = 6
CHUNK = 128
N_WORKERS = 32


def _fasttext_body(words_hbm, sw_hbm, wt_hbm, sp_hbm, out_hbm,
                   idxw_v, idxs_v, rows_v, spbuf_v, gsem, osem):
    wid = lax.axis_index("s") * 2 + lax.axis_index("c")
    per_worker = words_hbm.shape[0] // N_WORKERS
    n_chunks = per_worker // CHUNK

    pltpu.sync_copy(sp_hbm, spbuf_v)

    def chunk_base(c):
        return wid * per_worker + c * CHUNK

    def load_idx(c, b):
        base = chunk_base(c)
        pltpu.sync_copy(words_hbm.at[pl.ds(base, CHUNK)], idxw_v.at[b])
        pltpu.sync_copy(sw_hbm.at[pl.ds(base, CHUNK)], idxs_v.at[b])

    def gather(b, sb):
        return pltpu.make_async_copy(
            wt_hbm.at[idxw_v.at[b]], rows_v.at[b], gsem.at[sb])

    def outcopy(c, b, sb):
        return pltpu.make_async_copy(
            rows_v.at[b], out_hbm.at[pl.ds(chunk_base(c), CHUNK)], osem.at[sb])

    def add_special(b):
        @pl.loop(0, CHUNK // 16)
        def tok_group_body(tg):
            swv = idxs_v[b, pl.ds(tg * 16, 16)]
            swv = jnp.where(swv == 0, N_SPECIAL, swv)
            for k in range(16):
                sw = swv[k]
                t = tg * 16 + k

                @pl.when(sw != N_SPECIAL)
                def _():
                    for j in range(DIM_PAD // 16):
                        sp16 = spbuf_v[sw, pl.ds(j * 16, 16)]
                        plsc.addupdate(rows_v.at[b, t, pl.ds(j * 16, 16)],
                                       sp16)

    load_idx(0, 0)
    gather(0, 0).start()

    @pl.loop(0, n_chunks)
    def chunk_loop(c):
        b = lax.rem(c, 2)
        nb = 1 - b
        sb = lax.rem(c, 2)

        @pl.when(c + 1 < n_chunks)
        def _():
            load_idx(c + 1, nb)

            @pl.when(c >= 1)
            def _():
                outcopy(c - 1, nb, 1 - sb).wait()

            gather(nb, 1 - sb).start()

        gather(b, sb).wait()
        add_special(b)
        outcopy(c, b, sb).start()

    outcopy(n_chunks - 2, (n_chunks - 2) % 2, (n_chunks - 2) % 2).wait()
    outcopy(n_chunks - 1, (n_chunks - 1) % 2, (n_chunks - 1) % 2).wait()


_TBLK = 512


def _transpose_pad_block(wtt_ref, out_ref):
    x = wtt_ref[...]
    xt = jnp.swapaxes(x, 0, 1)
    out_ref[...] = jnp.pad(xt, ((0, 0), (0, DIM_PAD - DIM)))


def _transpose_pad(wt_t, n_rows):
    grid = (n_rows + _TBLK - 1) // _TBLK
    return pl.pallas_call(
        _transpose_pad_block,
        grid=(grid,),
        in_specs=[pl.BlockSpec((DIM, _TBLK), lambda i: (0, i))],
        out_specs=pl.BlockSpec((_TBLK, DIM_PAD), lambda i: (i, 0)),
        out_shape=jax.ShapeDtypeStruct((grid * _TBLK, DIM_PAD), jnp.float32),
    )(wt_t)


def kernel(words, special_words, word_table, special_table):
    B, S = words.shape
    words_f = words.reshape(-1).astype(jnp.int32)
    sw_f = special_words.reshape(-1).astype(jnp.int32)
    wt_pad = _transpose_pad(word_table.T, word_table.shape[0])
    sp_pad = jnp.pad(special_table, ((0, 0), (0, DIM_PAD - DIM)))
    run = pl.kernel(
        _fasttext_body,
        out_type=jax.ShapeDtypeStruct((B * S, DIM_PAD), jnp.float32),
        mesh=plsc.VectorSubcoreMesh(core_axis_name="c", subcore_axis_name="s"),
        scratch_types=[
            pltpu.VMEM((2, CHUNK), jnp.int32),
            pltpu.VMEM((2, CHUNK), jnp.int32),
            pltpu.VMEM((2, CHUNK, DIM_PAD), jnp.float32),
            pltpu.VMEM((N_SPECIAL + 1, DIM_PAD), jnp.float32),
            pltpu.SemaphoreType.DMA((2,)),
            pltpu.SemaphoreType.DMA((2,)),
        ],
    )
    out = run(words_f, sw_f, wt_pad, sp_pad)
    return out[:, :DIM].reshape(B, S, DIM)
